# baseline (speedup 1.0000x reference)
import functools

import jax
import jax.numpy as jnp
from jax import lax
from jax.experimental import pallas as pl
from jax.experimental.pallas import tpu as pltpu
from jax.experimental.pallas import tpu_sc as plsc

NC = 2
NS = 16
L = 16
NW = NC * NS


def kernel(idx, tok_emb, pos_emb):
    B, T = idx.shape
    V, D = tok_emb.shape
    N = B * T
    TW = T // NW
    CH = 32
    assert T % NW == 0 and D % L == 0 and TW % CH == 0 and B * TW >= 2 * CH

    chunks = []
    for b in range(B):
        for h in range(TW // CH):
            chunks.append((b * T + h * CH, b * TW + h * CH, h * CH))
    nch = len(chunks)

    mesh = plsc.VectorSubcoreMesh(
        core_axis_name="c", subcore_axis_name="s", num_cores=NC, num_subcores=NS
    )

    @functools.partial(
        pl.kernel,
        out_type=jax.ShapeDtypeStruct((N, D), jnp.float32),
        mesh=mesh,
        scratch_types=[
            pltpu.VMEM((B * TW,), jnp.int32),
            pltpu.VMEM((TW, D), jnp.float32),
            pltpu.VMEM((CH, D), jnp.float32),
            pltpu.VMEM((CH, D), jnp.float32),
            pltpu.SemaphoreType.DMA,
            pltpu.SemaphoreType.DMA,
            pltpu.SemaphoreType.DMA,
            pltpu.SemaphoreType.DMA,
        ],
    )
    def run(idx_hbm, tok_hbm, pos_hbm, out_hbm, idx_v, pos_v, rows0, rows1,
            g0, g1, s0, s1):
        wid = lax.axis_index("s") * NC + lax.axis_index("c")
        t0 = wid * TW
        rows = [rows0, rows1]
        gsem = [g0, g1]
        ssem = [s0, s1]

        for b in range(B):
            pltpu.sync_copy(idx_hbm.at[pl.ds(b * T + t0, TW)],
                            idx_v.at[pl.ds(b * TW, TW)])
        pltpu.sync_copy(pos_hbm.at[pl.ds(t0, TW)], pos_v)

        gd = [None, None]
        sd = [None, None]
        gd[0] = pltpu.async_copy(
            tok_hbm.at[idx_v.at[pl.ds(chunks[0][1], CH)]], rows[0], gsem[0])
        for g in range(nch):
            pb = g & 1
            nb = 1 - pb
            if g + 1 < nch:
                if sd[nb] is not None:
                    sd[nb].wait()
                gd[nb] = pltpu.async_copy(
                    tok_hbm.at[idx_v.at[pl.ds(chunks[g + 1][1], CH)]],
                    rows[nb], gsem[nb])
            gd[pb].wait()

            buf = rows[pb]
            po = chunks[g][2]

            @plsc.parallel_loop(0, CH, 1, unroll=4)
            def _(i, buf=buf, po=po):
                for j in range(D // L):
                    sl = (i, pl.ds(j * L, L))
                    buf[sl] = buf[sl] + pos_v[po + i, pl.ds(j * L, L)]

            sd[pb] = pltpu.async_copy(
                buf, out_hbm.at[pl.ds(t0 + chunks[g][0], CH)], ssem[pb])
        sd[0].wait()
        sd[1].wait()

    out = run(idx.reshape(-1), tok_emb, pos_emb)
    return out.reshape(B, T, D)


# speedup vs baseline: 1.0576x; 1.0576x over previous
import functools

import jax
import jax.numpy as jnp
from jax import lax
from jax.experimental import pallas as pl
from jax.experimental.pallas import tpu as pltpu
from jax.experimental.pallas import tpu_sc as plsc

NC = 2
NS = 16
L = 16
NW = NC * NS


def kernel(idx, tok_emb, pos_emb):
    B, T = idx.shape
    V, D = tok_emb.shape
    N = B * T
    TW = T // NW
    CH = 32
    assert T % NW == 0 and D % L == 0 and TW % CH == 0 and B * TW >= 2 * CH

    chunks = []
    for b in range(B):
        for h in range(TW // CH):
            chunks.append((b * T + h * CH, b * TW + h * CH, h * CH))
    nch = len(chunks)

    mesh = plsc.VectorSubcoreMesh(
        core_axis_name="c", subcore_axis_name="s", num_cores=NC, num_subcores=NS
    )

    @functools.partial(
        pl.kernel,
        out_type=jax.ShapeDtypeStruct((N, D), jnp.float32),
        mesh=mesh,
        scratch_types=[
            pltpu.VMEM((B * TW,), jnp.int32),
            pltpu.VMEM((TW, D), jnp.float32),
            pltpu.VMEM((CH, D), jnp.float32),
            pltpu.VMEM((CH, D), jnp.float32),
            pltpu.SemaphoreType.DMA,
            pltpu.SemaphoreType.DMA,
            pltpu.SemaphoreType.DMA,
            pltpu.SemaphoreType.DMA,
        ],
    )
    def run(idx_hbm, tok_hbm, pos_hbm, out_hbm, idx_v, pos_v, rows0, rows1,
            g0, g1, s0, s1):
        wid = lax.axis_index("s") * NC + lax.axis_index("c")
        t0 = wid * TW
        rows = [rows0, rows1]
        gsem = [g0, g1]
        ssem = [s0, s1]

        for b in range(B):
            pltpu.sync_copy(idx_hbm.at[pl.ds(b * T + t0, TW)],
                            idx_v.at[pl.ds(b * TW, TW)])
        pltpu.sync_copy(pos_hbm.at[pl.ds(t0, TW)], pos_v)

        gd = [None, None]
        sd = [None, None]
        gd[0] = pltpu.async_copy(
            tok_hbm.at[idx_v.at[pl.ds(chunks[0][1], CH)]], rows[0], gsem[0])
        for g in range(nch):
            pb = g & 1
            nb = 1 - pb
            if g + 1 < nch:
                if sd[nb] is not None:
                    sd[nb].wait()
                gd[nb] = pltpu.async_copy(
                    tok_hbm.at[idx_v.at[pl.ds(chunks[g + 1][1], CH)]],
                    rows[nb], gsem[nb])
            gd[pb].wait()

            buf = rows[pb]
            po = chunks[g][2]

            @plsc.parallel_loop(0, CH, 1, unroll=2)
            def _(i, buf=buf, po=po):
                for j in range(D // L):
                    sl = (i, pl.ds(j * L, L))
                    buf[sl] = buf[sl] + pos_v[po + i, pl.ds(j * L, L)]

            sd[pb] = pltpu.async_copy(
                buf, out_hbm.at[pl.ds(t0 + chunks[g][0], CH)], ssem[pb])
        sd[0].wait()
        sd[1].wait()

    out = run(idx.reshape(-1), tok_emb, pos_emb)
    return out.reshape(B, T, D)


# speedup vs baseline: 1.0847x; 1.0257x over previous
import functools

import jax
import jax.numpy as jnp
from jax import lax
from jax.experimental import pallas as pl
from jax.experimental.pallas import tpu as pltpu
from jax.experimental.pallas import tpu_sc as plsc

NC = 2
NS = 16
L = 16
NW = NC * NS


def kernel(idx, tok_emb, pos_emb):
    B, T = idx.shape
    V, D = tok_emb.shape
    N = B * T
    TW = T // NW
    CH = 32
    assert T % NW == 0 and D % L == 0 and TW % CH == 0 and B * TW >= 2 * CH

    chunks = []
    for b in range(B):
        for h in range(TW // CH):
            chunks.append((b * T + h * CH, b * TW + h * CH, h * CH))
    nch = len(chunks)

    mesh = plsc.VectorSubcoreMesh(
        core_axis_name="c", subcore_axis_name="s", num_cores=NC, num_subcores=NS
    )

    @functools.partial(
        pl.kernel,
        out_type=jax.ShapeDtypeStruct((N, D), jnp.float32),
        mesh=mesh,
        scratch_types=[
            pltpu.VMEM((B * TW,), jnp.int32),
            pltpu.VMEM((TW, D), jnp.float32),
            pltpu.VMEM((CH, D), jnp.float32),
            pltpu.VMEM((CH, D), jnp.float32),
            pltpu.SemaphoreType.DMA,
            pltpu.SemaphoreType.DMA,
            pltpu.SemaphoreType.DMA,
            pltpu.SemaphoreType.DMA,
        ],
    )
    def run(idx_hbm, tok_hbm, pos_hbm, out_hbm, idx_v, pos_v, rows0, rows1,
            g0, g1, s0, s1):
        wid = lax.axis_index("s") * NC + lax.axis_index("c")
        t0 = wid * TW
        rows = [rows0, rows1]
        gsem = [g0, g1]
        ssem = [s0, s1]

        for b in range(B):
            pltpu.sync_copy(idx_hbm.at[pl.ds(b * T + t0, TW)],
                            idx_v.at[pl.ds(b * TW, TW)])
        pltpu.sync_copy(pos_hbm.at[pl.ds(t0, TW)], pos_v)

        gd = [None, None]
        sd = [None, None]
        gd[0] = pltpu.async_copy(
            tok_hbm.at[idx_v.at[pl.ds(chunks[0][1], CH)]], rows[0], gsem[0])
        for g in range(nch):
            pb = g & 1
            nb = 1 - pb
            if g + 1 < nch:
                if sd[nb] is not None:
                    sd[nb].wait()
                gd[nb] = pltpu.async_copy(
                    tok_hbm.at[idx_v.at[pl.ds(chunks[g + 1][1], CH)]],
                    rows[nb], gsem[nb])
            gd[pb].wait()

            buf = rows[pb]
            po = chunks[g][2]

            @plsc.parallel_loop(0, CH, 1, unroll=2)
            def _(i, buf=buf, po=po):
                for j in range(D // L):
                    sl = (i, pl.ds(j * L, L))
                    plsc.addupdate(buf.at[sl], pos_v[po + i, pl.ds(j * L, L)])

            sd[pb] = pltpu.async_copy(
                buf, out_hbm.at[pl.ds(t0 + chunks[g][0], CH)], ssem[pb])
        sd[0].wait()
        sd[1].wait()

    out = run(idx.reshape(-1), tok_emb, pos_emb)
    return out.reshape(B, T, D)


# speedup vs baseline: 1.1991x; 1.1054x over previous
import functools

import jax
import jax.numpy as jnp
from jax import lax
from jax.experimental import pallas as pl
from jax.experimental.pallas import tpu as pltpu
from jax.experimental.pallas import tpu_sc as plsc

NC = 2
NS = 16
L = 16
NW = NC * NS


def kernel(idx, tok_emb, pos_emb):
    B, T = idx.shape
    V, D = tok_emb.shape
    N = B * T
    TW = T // NW
    CH = 32
    NB = 3
    assert T % NW == 0 and D % L == 0 and TW % CH == 0 and B * TW >= NB * CH

    chunks = []
    for b in range(B):
        for h in range(TW // CH):
            chunks.append((b * T + h * CH, b * TW + h * CH, h * CH))
    nch = len(chunks)

    mesh = plsc.VectorSubcoreMesh(
        core_axis_name="c", subcore_axis_name="s", num_cores=NC, num_subcores=NS
    )

    @functools.partial(
        pl.kernel,
        out_type=jax.ShapeDtypeStruct((N, D), jnp.float32),
        mesh=mesh,
        scratch_types=[
            pltpu.VMEM((B * TW,), jnp.int32),
            pltpu.VMEM((TW, D), jnp.float32),
            [pltpu.VMEM((CH, D), jnp.float32)] * 3,
            [pltpu.SemaphoreType.DMA] * 3,
            [pltpu.SemaphoreType.DMA] * 3,
            pltpu.SemaphoreType.DMA,
            pltpu.SemaphoreType.DMA,
        ],
    )
    def run(idx_hbm, tok_hbm, pos_hbm, out_hbm, idx_v, pos_v, rows,
            gsem, ssem, isem, psem):
        wid = lax.axis_index("s") * NC + lax.axis_index("c")
        t0 = wid * TW

        idone = [
            pltpu.async_copy(idx_hbm.at[pl.ds(b * T + t0, TW)],
                             idx_v.at[pl.ds(b * TW, TW)], isem)
            for b in range(B)
        ]
        pdone = pltpu.async_copy(pos_hbm.at[pl.ds(t0, TW)], pos_v, psem)
        for d in idone:
            d.wait()

        def gather(g):
            b = g % NB
            return pltpu.async_copy(
                tok_hbm.at[idx_v.at[pl.ds(chunks[g][1], CH)]],
                rows[b], gsem[b])

        gd = [None] * NB
        sd = [None] * NB
        gd[0] = gather(0)
        gd[1] = gather(1)
        pdone.wait()
        for g in range(nch):
            b = g % NB
            gd[b].wait()

            buf = rows[b]
            po = chunks[g][2]

            @plsc.parallel_loop(0, CH, 1, unroll=2)
            def _(i, buf=buf, po=po):
                for j in range(D // L):
                    sl = (i, pl.ds(j * L, L))
                    plsc.addupdate(buf.at[sl], pos_v[po + i, pl.ds(j * L, L)])

            sd[b] = pltpu.async_copy(
                buf, out_hbm.at[pl.ds(t0 + chunks[g][0], CH)], ssem[b])
            if g + 2 < nch:
                nb = (g + 2) % NB
                if sd[nb] is not None:
                    sd[nb].wait()
                gd[nb] = gather(g + 2)
        for d in sd:
            if d is not None:
                d.wait()

    out = run(idx.reshape(-1), tok_emb, pos_emb)
    return out.reshape(B, T, D)


# speedup vs baseline: 1.2036x; 1.0038x over previous
import functools

import jax
import jax.numpy as jnp
from jax import lax
from jax.experimental import pallas as pl
from jax.experimental.pallas import tpu as pltpu
from jax.experimental.pallas import tpu_sc as plsc

NC = 2
NS = 16
L = 16
NW = NC * NS


def kernel(idx, tok_emb, pos_emb):
    B, T = idx.shape
    V, D = tok_emb.shape
    N = B * T
    TW = T // NW
    CH = 32
    NB = 3
    assert T % NW == 0 and D % L == 0 and TW % CH == 0 and B * TW >= NB * CH

    chunks = []
    for b in range(B):
        for h in range(TW // CH):
            chunks.append((b, b * TW + h * CH, h * CH))
    nch = len(chunks)

    mesh = plsc.VectorSubcoreMesh(
        core_axis_name="c", subcore_axis_name="s", num_cores=NC, num_subcores=NS
    )

    @functools.partial(
        pl.kernel,
        out_type=jax.ShapeDtypeStruct((B, T, D), jnp.float32),
        mesh=mesh,
        scratch_types=[
            pltpu.VMEM((B * TW,), jnp.int32),
            pltpu.VMEM((TW, D), jnp.float32),
            [pltpu.VMEM((CH, D), jnp.float32)] * NB,
            [pltpu.SemaphoreType.DMA] * NB,
            [pltpu.SemaphoreType.DMA] * NB,
            pltpu.SemaphoreType.DMA,
            pltpu.SemaphoreType.DMA,
        ],
    )
    def run(idx_hbm, tok_hbm, pos_hbm, out_hbm, idx_v, pos_v, rows,
            gsem, ssem, isem, psem):
        wid = lax.axis_index("s") * NC + lax.axis_index("c")
        t0 = wid * TW

        idone = [
            pltpu.async_copy(idx_hbm.at[b, pl.ds(t0, TW)],
                             idx_v.at[pl.ds(b * TW, TW)], isem)
            for b in range(B)
        ]
        pdone = pltpu.async_copy(pos_hbm.at[pl.ds(t0, TW)], pos_v, psem)
        for d in idone:
            d.wait()

        def gather(g):
            b = g % NB
            return pltpu.async_copy(
                tok_hbm.at[idx_v.at[pl.ds(chunks[g][1], CH)]],
                rows[b], gsem[b])

        gd = [None] * NB
        sd = [None] * NB
        LA = NB - 1
        for k in range(min(LA, nch)):
            gd[k % NB] = gather(k)
        pdone.wait()
        for g in range(nch):
            b = g % NB
            gd[b].wait()

            buf = rows[b]
            po = chunks[g][2]

            @plsc.parallel_loop(0, CH, 1, unroll=2)
            def _(i, buf=buf, po=po):
                for j in range(D // L):
                    sl = (i, pl.ds(j * L, L))
                    plsc.addupdate(buf.at[sl], pos_v[po + i, pl.ds(j * L, L)])

            sd[b] = pltpu.async_copy(
                buf, out_hbm.at[chunks[g][0], pl.ds(t0 + chunks[g][2], CH)],
                ssem[b])
            if g + LA < nch:
                nb = (g + LA) % NB
                if sd[nb] is not None:
                    sd[nb].wait()
                gd[nb] = gather(g + LA)
        for d in sd:
            if d is not None:
                d.wait()

    return run(idx, tok_emb, pos_emb)
